# baseline (speedup 1.0000x reference)
import functools

import jax
import jax.numpy as jnp
from jax import lax
from jax.experimental import pallas as pl
from jax.experimental.pallas import tpu as pltpu
from jax.experimental.pallas import tpu_sc as plsc

_NC = 2
_NS = 16
_NW = _NC * _NS
_LANES = 16
_ZUNROLL = 8


def _build_a_sc(topo_hbm, w_hbm, a_hbm, topo_v, w_v, buf, sem0, sem1,
                *, S, ROWS_W, CH):
    wid = lax.axis_index("s") * _NC + lax.axis_index("c")
    base = wid * ROWS_W
    pltpu.sync_copy(topo_hbm.at[pl.ds(base, ROWS_W)], topo_v)
    pltpu.sync_copy(w_hbm.at[pl.ds(base, ROWS_W)], w_v)
    zero16 = jnp.zeros((_LANES,), jnp.float32)
    sems = [sem0, sem1]
    pending = [None, None]
    half = CH * S
    for ch in range(ROWS_W // CH):
        bsel = ch & 1
        boff = bsel * half
        if pending[bsel] is not None:
            pending[bsel].wait()
            pending[bsel] = None

        def zbody(i, carry, boff=boff):
            for u in range(_ZUNROLL):
                buf[pl.ds(boff + (i * _ZUNROLL + u) * _LANES, _LANES)] = zero16
            return carry
        lax.fori_loop(0, half // (_LANES * _ZUNROLL), zbody, 0)

        for r in range(CH):
            row = ch * CH + r
            plsc.addupdate_scatter(buf, [topo_v[row] + (boff + r * S)],
                                   w_v[row])

        pending[bsel] = pltpu.async_copy(
            buf.at[pl.ds(boff, half)],
            a_hbm.at[pl.ds((base + ch * CH) * S, half)],
            sems[bsel])
    for p in pending:
        if p is not None:
            p.wait()


def _sc_build_a(topo_flat, w_flat, S):
    rows = topo_flat.shape[0]
    rows_w = rows // _NW
    ch = 16
    while rows_w % ch:
        ch //= 2
    builder = pl.kernel(
        functools.partial(_build_a_sc, S=S, ROWS_W=rows_w, CH=ch),
        out_type=jax.ShapeDtypeStruct((rows * S,), jnp.float32),
        mesh=plsc.VectorSubcoreMesh(core_axis_name="c", subcore_axis_name="s"),
        compiler_params=pltpu.CompilerParams(needs_layout_passes=False),
        scratch_types=[
            pltpu.VMEM((rows_w, _LANES), jnp.int32),
            pltpu.VMEM((rows_w, _LANES), jnp.float32),
            pltpu.VMEM((2 * ch * S,), jnp.float32),
            pltpu.SemaphoreType.DMA,
            pltpu.SemaphoreType.DMA,
        ],
    )
    return builder(topo_flat, w_flat)


def _fold_kernel(Wm_ref, Wa_ref, bm_ref, WmWa_ref, bmWa_ref):
    WmWa_ref[...] = jnp.dot(Wm_ref[...], Wa_ref[...],
                            preferred_element_type=jnp.float32).astype(jnp.bfloat16)
    bmWa_ref[...] = jnp.dot(bm_ref[...], Wa_ref[...],
                            preferred_element_type=jnp.float32)


def _bf(v):
    return v.astype(jnp.bfloat16)


def _dense_chain(x_b, aggx, sumw, WmWa_ref, bmWa_ref, ba_ref, Wg_ref, bg_ref,
                 Wo_ref, bo_ref, out_ref, DIM):
    x16 = _bf(x_b)
    agg = (jnp.dot(_bf(aggx), WmWa_ref[...], preferred_element_type=jnp.float32)
           + sumw * bmWa_ref[...] + ba_ref[...])
    agg16 = _bf(agg)
    gate_logits = (jnp.dot(x16, Wg_ref[:DIM], preferred_element_type=jnp.float32)
                   + jnp.dot(agg16, Wg_ref[DIM:], preferred_element_type=jnp.float32)
                   + bg_ref[...])
    g = jax.nn.sigmoid(gate_logits)
    upd = x_b + g * (agg - x_b)
    out_ref[0] = jnp.dot(_bf(upd), Wo_ref[...],
                         preferred_element_type=jnp.float32) + bo_ref[...]


def _main_kernel_a(x_full_ref, a_ref, w_ref,
                   WmWa_ref, bmWa_ref, ba_ref, Wg_ref, bg_ref, Wo_ref, bo_ref,
                   out_ref, *, DIM, T):
    b = pl.program_id(0)
    t = pl.program_id(1)
    aggx = jnp.dot(a_ref[0], x_full_ref[b],
                   preferred_element_type=jnp.float32)
    x_b = x_full_ref[b, pl.ds(t * T, T)]
    sumw = jnp.sum(w_ref[0], axis=1, keepdims=True)
    _dense_chain(x_b, aggx, sumw, WmWa_ref, bmWa_ref, ba_ref,
                 Wg_ref, bg_ref, Wo_ref, bo_ref, out_ref, DIM)


def _main_kernel_onehot(x_blk_ref, x_full_ref, topo_ref, w_ref,
                        WmWa_ref, bmWa_ref, ba_ref, Wg_ref, bg_ref, Wo_ref,
                        bo_ref, out_ref, *, S, K, T, DIM):
    topo = topo_ref[0]
    w = w_ref[0]
    iota = lax.broadcasted_iota(jnp.int32, (T, S), 1)
    A = jnp.zeros((T, S), dtype=jnp.float32)
    for k in range(K):
        A = A + jnp.where(topo[:, k][:, None] == iota, w[:, k][:, None], 0.0)
    aggx = jnp.dot(A, x_full_ref[0], preferred_element_type=jnp.float32)
    sumw = jnp.sum(w, axis=1, keepdims=True)
    _dense_chain(x_blk_ref[0], aggx, sumw, WmWa_ref, bmWa_ref, ba_ref,
                 Wg_ref, bg_ref, Wo_ref, bo_ref, out_ref, DIM)


def kernel(x, topology, weights, Wm, bm, Wa, ba, Wg, bg, Wo, bo):
    B, S, DIM = x.shape
    K = topology.shape[-1]
    T = min(512, S)

    bm2 = bm.reshape(1, DIM)
    ba2 = ba.reshape(1, DIM)
    bg2 = bg.reshape(1, DIM)
    bo2 = bo.reshape(1, DIM)

    WmWa, bmWa = pl.pallas_call(
        _fold_kernel,
        out_shape=(jax.ShapeDtypeStruct((DIM, DIM), jnp.bfloat16),
                   jax.ShapeDtypeStruct((1, DIM), jnp.float32)),
    )(Wm, Wa, bm2)
    Wg16 = Wg.astype(jnp.bfloat16)
    Wo16 = Wo.astype(jnp.bfloat16)

    rows = B * S
    use_sc = (K == _LANES and rows % _NW == 0
              and S % (_LANES * _ZUNROLL) == 0 and (rows // _NW) % 2 == 0)

    grid = (B, S // T)
    common_specs = [
        pl.BlockSpec((DIM, DIM), lambda b, t: (0, 0)),
        pl.BlockSpec((1, DIM), lambda b, t: (0, 0)),
        pl.BlockSpec((1, DIM), lambda b, t: (0, 0)),
        pl.BlockSpec((2 * DIM, DIM), lambda b, t: (0, 0)),
        pl.BlockSpec((1, DIM), lambda b, t: (0, 0)),
        pl.BlockSpec((DIM, DIM), lambda b, t: (0, 0)),
        pl.BlockSpec((1, DIM), lambda b, t: (0, 0)),
    ]
    common_args = (WmWa, bmWa, ba2, Wg16, bg2, Wo16, bo2)

    if use_sc:
        a_flat = _sc_build_a(topology.reshape(rows, K),
                             weights.reshape(rows, K), S)
        A = a_flat.reshape(B, S, S)
        out = pl.pallas_call(
            functools.partial(_main_kernel_a, DIM=DIM, T=T),
            grid=grid,
            in_specs=[
                pl.BlockSpec((B, S, DIM), lambda b, t: (0, 0, 0)),
                pl.BlockSpec((1, T, S), lambda b, t: (b, t, 0)),
                pl.BlockSpec((1, T, K), lambda b, t: (b, t, 0)),
                *common_specs,
            ],
            out_specs=pl.BlockSpec((1, T, DIM), lambda b, t: (b, t, 0)),
            out_shape=jax.ShapeDtypeStruct((B, S, DIM), jnp.float32),
        )(x, A, weights, *common_args)
    else:
        out = pl.pallas_call(
            functools.partial(_main_kernel_onehot, S=S, K=K, T=T, DIM=DIM),
            grid=grid,
            in_specs=[
                pl.BlockSpec((1, T, DIM), lambda b, t: (b, t, 0)),
                pl.BlockSpec((1, S, DIM), lambda b, t: (b, 0, 0)),
                pl.BlockSpec((1, T, K), lambda b, t: (b, t, 0)),
                pl.BlockSpec((1, T, K), lambda b, t: (b, t, 0)),
                *common_specs,
            ],
            out_specs=pl.BlockSpec((1, T, DIM), lambda b, t: (b, t, 0)),
            out_shape=jax.ShapeDtypeStruct((B, S, DIM), jnp.float32),
        )(x, x, topology, weights, *common_args)
    return out


# speedup vs baseline: 1.3805x; 1.3805x over previous
import functools

import jax
import jax.numpy as jnp
from jax import lax
from jax.experimental import pallas as pl
from jax.experimental.pallas import tpu as pltpu
from jax.experimental.pallas import tpu_sc as plsc

_NC = 2
_NS = 16
_NW = _NC * _NS
_LANES = 16
_ZUNROLL = 8


def _build_a_sc(topo_hbm, w_hbm, a_hbm, topo_v, w_v, buf, sem0, sem1,
                *, S, ROWS_W, CH):
    wid = lax.axis_index("s") * _NC + lax.axis_index("c")
    base = wid * ROWS_W
    pltpu.sync_copy(topo_hbm.at[pl.ds(base, ROWS_W)], topo_v)
    pltpu.sync_copy(w_hbm.at[pl.ds(base, ROWS_W)], w_v)
    zero16 = jnp.zeros((_LANES,), jnp.float32)
    sems = [sem0, sem1]
    pending = [None, None]
    for ch in range(ROWS_W // CH):
        bsel = ch & 1
        r0 = bsel * CH
        if pending[bsel] is not None:
            pending[bsel].wait()
            pending[bsel] = None

        def zouter(r, carry):
            def zinner(i, c2):
                for u in range(_ZUNROLL):
                    buf[r, pl.ds((i * _ZUNROLL + u) * _LANES, _LANES)] = zero16
                return c2
            return lax.fori_loop(0, S // (_LANES * _ZUNROLL), zinner, carry)
        lax.fori_loop(r0, r0 + CH, zouter, 0)

        for r in range(CH):
            row = ch * CH + r
            plsc.addupdate_scatter(
                buf, [jnp.full((_LANES,), r0 + r, jnp.int32), topo_v[row]],
                w_v[row])

        pending[bsel] = pltpu.async_copy(
            buf.at[pl.ds(r0, CH)],
            a_hbm.at[pl.ds(base + ch * CH, CH)],
            sems[bsel])
    for p in pending:
        if p is not None:
            p.wait()


def _sc_build_a(topo_flat, w_flat, S):
    rows = topo_flat.shape[0]
    rows_w = rows // _NW
    ch = 16
    while rows_w % ch:
        ch //= 2
    builder = pl.kernel(
        functools.partial(_build_a_sc, S=S, ROWS_W=rows_w, CH=ch),
        out_type=jax.ShapeDtypeStruct((rows, S), jnp.float32),
        mesh=plsc.VectorSubcoreMesh(core_axis_name="c", subcore_axis_name="s"),
        compiler_params=pltpu.CompilerParams(needs_layout_passes=False),
        scratch_types=[
            pltpu.VMEM((rows_w, _LANES), jnp.int32),
            pltpu.VMEM((rows_w, _LANES), jnp.float32),
            pltpu.VMEM((2 * ch, S), jnp.float32),
            pltpu.SemaphoreType.DMA,
            pltpu.SemaphoreType.DMA,
        ],
    )
    return builder(topo_flat, w_flat)


def _fold_kernel(Wm_ref, Wa_ref, bm_ref, WmWa_ref, bmWa_ref):
    WmWa_ref[...] = jnp.dot(Wm_ref[...], Wa_ref[...],
                            preferred_element_type=jnp.float32)
    bmWa_ref[...] = jnp.dot(bm_ref[...], Wa_ref[...],
                            preferred_element_type=jnp.float32)


def _bf(v):
    return v.astype(jnp.bfloat16)


def _dense_chain(x_b, aggx, sumw, WmWa_ref, bmWa_ref, ba_ref, Wg_ref, bg_ref,
                 Wo_ref, bo_ref, out_ref, DIM):
    agg = (jnp.dot(aggx, WmWa_ref[...], preferred_element_type=jnp.float32)
           + sumw * bmWa_ref[...] + ba_ref[...])
    gate_logits = (jnp.dot(x_b, Wg_ref[:DIM], preferred_element_type=jnp.float32)
                   + jnp.dot(agg, Wg_ref[DIM:], preferred_element_type=jnp.float32)
                   + bg_ref[...])
    g = jax.nn.sigmoid(gate_logits)
    upd = x_b + g * (agg - x_b)
    out_ref[0] = jnp.dot(upd, Wo_ref[...],
                         preferred_element_type=jnp.float32) + bo_ref[...]


def _main_kernel_a(x_blk_ref, x_full_ref, a_ref, w_ref,
                   WmWa_ref, bmWa_ref, ba_ref, Wg_ref, bg_ref, Wo_ref, bo_ref,
                   out_ref, *, DIM):
    aggx = jnp.dot(a_ref[...], x_full_ref[0],
                   preferred_element_type=jnp.float32)
    sumw = jnp.sum(w_ref[0], axis=1, keepdims=True)
    _dense_chain(x_blk_ref[0], aggx, sumw, WmWa_ref, bmWa_ref, ba_ref,
                 Wg_ref, bg_ref, Wo_ref, bo_ref, out_ref, DIM)


def _main_kernel_onehot(x_blk_ref, x_full_ref, topo_ref, w_ref,
                        WmWa_ref, bmWa_ref, ba_ref, Wg_ref, bg_ref, Wo_ref,
                        bo_ref, out_ref, *, S, K, T, DIM):
    topo = topo_ref[0]
    w = w_ref[0]
    iota = lax.broadcasted_iota(jnp.int32, (T, S), 1)
    A = jnp.zeros((T, S), dtype=jnp.float32)
    for k in range(K):
        A = A + jnp.where(topo[:, k][:, None] == iota, w[:, k][:, None], 0.0)
    aggx = jnp.dot(A, x_full_ref[0], preferred_element_type=jnp.float32)
    sumw = jnp.sum(w, axis=1, keepdims=True)
    _dense_chain(x_blk_ref[0], aggx, sumw, WmWa_ref, bmWa_ref, ba_ref,
                 Wg_ref, bg_ref, Wo_ref, bo_ref, out_ref, DIM)


def kernel(x, topology, weights, Wm, bm, Wa, ba, Wg, bg, Wo, bo):
    B, S, DIM = x.shape
    K = topology.shape[-1]
    T = min(512, S)

    bm2 = bm.reshape(1, DIM)
    ba2 = ba.reshape(1, DIM)
    bg2 = bg.reshape(1, DIM)
    bo2 = bo.reshape(1, DIM)

    WmWa, bmWa = pl.pallas_call(
        _fold_kernel,
        out_shape=(jax.ShapeDtypeStruct((DIM, DIM), jnp.float32),
                   jax.ShapeDtypeStruct((1, DIM), jnp.float32)),
    )(Wm, Wa, bm2)

    rows = B * S
    use_sc = (K == _LANES and rows % _NW == 0
              and S % (_LANES * _ZUNROLL) == 0 and (rows // _NW) % 2 == 0)

    grid = (B, S // T)
    common_specs = [
        pl.BlockSpec((DIM, DIM), lambda b, t: (0, 0)),
        pl.BlockSpec((1, DIM), lambda b, t: (0, 0)),
        pl.BlockSpec((1, DIM), lambda b, t: (0, 0)),
        pl.BlockSpec((2 * DIM, DIM), lambda b, t: (0, 0)),
        pl.BlockSpec((1, DIM), lambda b, t: (0, 0)),
        pl.BlockSpec((DIM, DIM), lambda b, t: (0, 0)),
        pl.BlockSpec((1, DIM), lambda b, t: (0, 0)),
    ]
    common_args = (WmWa, bmWa, ba2, Wg, bg2, Wo, bo2)

    if use_sc:
        A = _sc_build_a(topology.reshape(rows, K),
                        weights.reshape(rows, K), S)
        nt = S // T
        out = pl.pallas_call(
            functools.partial(_main_kernel_a, DIM=DIM),
            grid=grid,
            in_specs=[
                pl.BlockSpec((1, T, DIM), lambda b, t: (b, t, 0)),
                pl.BlockSpec((1, S, DIM), lambda b, t: (b, 0, 0)),
                pl.BlockSpec((T, S), lambda b, t: (b * nt + t, 0)),
                pl.BlockSpec((1, T, K), lambda b, t: (b, t, 0)),
                *common_specs,
            ],
            out_specs=pl.BlockSpec((1, T, DIM), lambda b, t: (b, t, 0)),
            out_shape=jax.ShapeDtypeStruct((B, S, DIM), jnp.float32),
        )(x, x, A, weights, *common_args)
    else:
        out = pl.pallas_call(
            functools.partial(_main_kernel_onehot, S=S, K=K, T=T, DIM=DIM),
            grid=grid,
            in_specs=[
                pl.BlockSpec((1, T, DIM), lambda b, t: (b, t, 0)),
                pl.BlockSpec((1, S, DIM), lambda b, t: (b, 0, 0)),
                pl.BlockSpec((1, T, K), lambda b, t: (b, t, 0)),
                pl.BlockSpec((1, T, K), lambda b, t: (b, t, 0)),
                *common_specs,
            ],
            out_specs=pl.BlockSpec((1, T, DIM), lambda b, t: (b, t, 0)),
            out_shape=jax.ShapeDtypeStruct((B, S, DIM), jnp.float32),
        )(x, x, topology, weights, *common_args)
    return out
